# baseline (speedup 1.0000x reference)
import functools

import jax
import jax.numpy as jnp
from jax import lax
from jax.experimental import pallas as pl
from jax.experimental.pallas import tpu as pltpu
from jax.experimental.pallas import tpu_sc as plsc

D = 128
SEQ = 200
BATCH = 1024
L = 16

NC = 2
NS = 16
NW = NC * NS
B_PER_W = BATCH // NW

NBUF = 4

_CHUNKS = ((0, 128), (128, 72))


def _emb_body(x_hbm, tok_hbm, pos_hbm, out_hbm, pos_v, idxs, rows, sem_i,
              sem_g, sem_w):
    wid = lax.axis_index("s") * NC + lax.axis_index("c")
    base = wid * B_PER_W
    pos_cp = pltpu.async_copy(pos_hbm, pos_v, sem_w)

    def fire_idx(i):
        return pltpu.async_copy(x_hbm.at[base + i], idxs[i % NBUF], sem_i)

    def fire_gathers(i):
        buf = i % NBUF
        return [
            pltpu.async_copy(
                tok_hbm.at[idxs[buf].at[pl.ds(off, n)]],
                rows[buf].at[pl.ds(off, n)],
                sem_g,
            )
            for off, n in _CHUNKS
        ]

    def add_pos(i):
        buf = i % NBUF

        def row_body(r2, c):
            for u in range(2):
                r = 2 * r2 + u
                for j in range(D // L):
                    sl = pl.ds(j * L, L)
                    rows[buf][r, sl] = rows[buf][r, sl] + pos_v[r, sl]
            return c

        lax.fori_loop(0, SEQ // 2, row_body, 0)

    def fire_write(i):
        return pltpu.async_copy(rows[i % NBUF], out_hbm.at[base + i], sem_w)

    idx_cps = {i: fire_idx(i) for i in range(min(3, B_PER_W))}
    gather_cps = {}
    write_cps = {}
    for i in range(min(2, B_PER_W)):
        idx_cps.pop(i).wait()
        gather_cps[i] = fire_gathers(i)
    pos_cp.wait()

    for i in range(B_PER_W):
        if i + 3 < B_PER_W:
            idx_cps[i + 3] = fire_idx(i + 3)
        if i + 2 < B_PER_W:
            if i - 2 in write_cps:
                write_cps.pop(i - 2).wait()
            idx_cps.pop(i + 2).wait()
            gather_cps[i + 2] = fire_gathers(i + 2)
        for cp in gather_cps.pop(i):
            cp.wait()
        add_pos(i)
        write_cps[i] = fire_write(i)

    for i in sorted(write_cps):
        write_cps.pop(i).wait()


@functools.partial(
    pl.kernel,
    mesh=plsc.VectorSubcoreMesh(core_axis_name="c", subcore_axis_name="s"),
    out_type=jax.ShapeDtypeStruct((BATCH, SEQ, D), jnp.float32),
    scratch_types=[
        pltpu.VMEM((SEQ, D), jnp.float32),
        [pltpu.VMEM((SEQ,), jnp.int32) for _ in range(NBUF)],
        [pltpu.VMEM((SEQ, D), jnp.float32) for _ in range(NBUF)],
        pltpu.SemaphoreType.DMA,
        pltpu.SemaphoreType.DMA,
        pltpu.SemaphoreType.DMA,
    ],
)
def _emb_kernel(x_hbm, tok_hbm, pos_hbm, out_hbm, pos_v, idxs, rows, sem_i,
                sem_g, sem_w):
    _emb_body(x_hbm, tok_hbm, pos_hbm, out_hbm, pos_v, idxs, rows, sem_i,
              sem_g, sem_w)


def kernel(x, token_table, pos_table):
    return _emb_kernel(x, token_table, pos_table)


# speedup vs baseline: 1.0502x; 1.0502x over previous
import functools

import jax
import jax.numpy as jnp
from jax import lax
from jax.experimental import pallas as pl
from jax.experimental.pallas import tpu as pltpu
from jax.experimental.pallas import tpu_sc as plsc

D = 128
SEQ = 200
BATCH = 1024
L = 16

NC = 2
NS = 16
NW = NC * NS
B_PER_W = BATCH // NW

NBUF = 4

_CHUNKS = ((0, 128), (128, 72))


def _emb_body(x_hbm, tok_hbm, pos_hbm, out_hbm, pos_v, idxs, rows, sem_i,
              sem_g, sem_w):
    wid = lax.axis_index("s") * NC + lax.axis_index("c")
    base = wid * B_PER_W
    pos_cp = pltpu.async_copy(pos_hbm, pos_v, sem_w)

    def fire_idx(i, buf):
        pltpu.async_copy(x_hbm.at[base + i], idxs[buf], sem_i)

    def wait_idx(buf):
        pltpu.make_async_copy(x_hbm.at[base], idxs[buf], sem_i).wait()

    def fire_gathers(buf):
        for off, n in _CHUNKS:
            pltpu.async_copy(
                tok_hbm.at[idxs[buf].at[pl.ds(off, n)]],
                rows[buf].at[pl.ds(off, n)],
                sem_g,
            )

    def wait_gathers(buf):
        for off, n in _CHUNKS:
            pltpu.make_async_copy(
                tok_hbm.at[idxs[buf].at[pl.ds(off, n)]],
                rows[buf].at[pl.ds(off, n)],
                sem_g,
            ).wait()

    def add_pos(buf):
        def row_body(r, c):
            for j in range(D // L):
                sl = pl.ds(j * L, L)
                rows[buf][r, sl] = rows[buf][r, sl] + pos_v[r, sl]
            return c

        lax.fori_loop(0, SEQ, row_body, 0)

    def fire_write(i, buf):
        pltpu.async_copy(rows[buf], out_hbm.at[base + i], sem_w)

    def wait_write():
        pltpu.make_async_copy(rows[0], out_hbm.at[base], sem_w).wait()

    def step(i, par, steady):
        if steady or i + 3 < B_PER_W:
            fire_idx(i + 3, (par + 3) % NBUF)
        if steady or i + 2 < B_PER_W:
            if steady or i - 2 >= 0:
                wait_write()
            wait_idx((par + 2) % NBUF)
            fire_gathers((par + 2) % NBUF)
        wait_gathers(par)
        add_pos(par)
        fire_write(i, par)

    for i in range(3):
        fire_idx(i, i)
    for i in range(2):
        wait_idx(i)
        fire_gathers(i)
    pos_cp.wait()

    for i in range(4):
        step(i, i, False)

    def steady_body(g, c):
        for par in range(NBUF):
            step(g * NBUF + par, par, True)
        return c

    lax.fori_loop(1, B_PER_W // NBUF - 1, steady_body, 0)

    for i in range(B_PER_W - 4, B_PER_W):
        step(i, i % NBUF, False)
    for _ in range(4):
        wait_write()


@functools.partial(
    pl.kernel,
    mesh=plsc.VectorSubcoreMesh(core_axis_name="c", subcore_axis_name="s"),
    out_type=jax.ShapeDtypeStruct((BATCH, SEQ, D), jnp.float32),
    scratch_types=[
        pltpu.VMEM((SEQ, D), jnp.float32),
        [pltpu.VMEM((SEQ,), jnp.int32) for _ in range(NBUF)],
        [pltpu.VMEM((SEQ, D), jnp.float32) for _ in range(NBUF)],
        pltpu.SemaphoreType.DMA,
        pltpu.SemaphoreType.DMA,
        pltpu.SemaphoreType.DMA,
    ],
)
def _emb_kernel(x_hbm, tok_hbm, pos_hbm, out_hbm, pos_v, idxs, rows, sem_i,
                sem_g, sem_w):
    _emb_body(x_hbm, tok_hbm, pos_hbm, out_hbm, pos_v, idxs, rows, sem_i,
              sem_g, sem_w)


def kernel(x, token_table, pos_table):
    return _emb_kernel(x, token_table, pos_table)
